# baseline (speedup 1.0000x reference)
import functools
import jax
import jax.numpy as jnp
from jax import lax
from jax.experimental import pallas as pl
from jax.experimental.pallas import tpu as pltpu
from jax.experimental.pallas import tpu_sc as plsc

BATCH = 4096
SEQ = 200
H_DIM = 32
LANES = 128
TAB_ROWS = SEQ * H_DIM // LANES
OUT_ROWS = BATCH * SEQ * H_DIM // LANES

_INFO = plsc.get_sparse_core_info()
NC, NS = _INFO.num_cores, _INFO.num_subcores
NW = NC * NS
ROWS_PER_W = OUT_ROWS // NW
REP = 16
REP_ROWS = REP * TAB_ROWS
NDMA = ROWS_PER_W // REP_ROWS


def _sc_body(table_hbm, out_hbm, rep_v, sem):
    wid = lax.axis_index("s") * NC + lax.axis_index("c")
    base = wid * ROWS_PER_W
    for r in range(REP):
        pltpu.async_copy(table_hbm, rep_v.at[pl.ds(r * TAB_ROWS, TAB_ROWS)], sem)
    for r in range(REP):
        pltpu.make_async_copy(
            table_hbm, rep_v.at[pl.ds(r * TAB_ROWS, TAB_ROWS)], sem
        ).wait()
    for j in range(NDMA):
        pltpu.async_copy(
            rep_v, out_hbm.at[pl.ds(base + j * REP_ROWS, REP_ROWS)], sem
        )
    for j in range(NDMA):
        pltpu.make_async_copy(
            rep_v, out_hbm.at[pl.ds(base + j * REP_ROWS, REP_ROWS)], sem
        ).wait()


_sc_call = functools.partial(
    pl.kernel,
    mesh=plsc.VectorSubcoreMesh(core_axis_name="c", subcore_axis_name="s"),
    out_type=jax.ShapeDtypeStruct((OUT_ROWS, LANES), jnp.float32),
    scratch_types=[
        pltpu.VMEM((REP_ROWS, LANES), jnp.float32),
        pltpu.SemaphoreType.DMA,
    ],
)(_sc_body)


def kernel(x, pos_embedding):
    del x
    out = _sc_call(pos_embedding[:SEQ].reshape(TAB_ROWS, LANES))
    return out.reshape(BATCH, SEQ, H_DIM)


# speedup vs baseline: 2.8101x; 2.8101x over previous
import functools
import jax
import jax.numpy as jnp
from jax import lax
from jax.experimental import pallas as pl
from jax.experimental.pallas import tpu as pltpu
from jax.experimental.pallas import tpu_sc as plsc

BATCH = 4096
SEQ = 200
H_DIM = 32
ROW = SEQ * H_DIM

_INFO = plsc.get_sparse_core_info()
NC, NS = _INFO.num_cores, _INFO.num_subcores
NW = NC * NS
ROWS_PER_W = BATCH // NW
REP = 16
NDMA = ROWS_PER_W // REP


def _sc_body(table_hbm, out_hbm, rep_v, sem):
    wid = lax.axis_index("s") * NC + lax.axis_index("c")
    base = wid * ROWS_PER_W
    for r in range(REP):
        pltpu.async_copy(table_hbm, rep_v.at[pl.ds(r, 1)], sem)
    for r in range(REP):
        pltpu.make_async_copy(table_hbm, rep_v.at[pl.ds(r, 1)], sem).wait()
    for j in range(NDMA):
        pltpu.async_copy(rep_v, out_hbm.at[pl.ds(base + j * REP, REP)], sem)
    for j in range(NDMA):
        pltpu.make_async_copy(
            rep_v, out_hbm.at[pl.ds(base + j * REP, REP)], sem
        ).wait()


_sc_call = functools.partial(
    pl.kernel,
    mesh=plsc.VectorSubcoreMesh(core_axis_name="c", subcore_axis_name="s"),
    out_type=jax.ShapeDtypeStruct((BATCH, ROW), jnp.float32),
    scratch_types=[
        pltpu.VMEM((REP, ROW), jnp.float32),
        pltpu.SemaphoreType.DMA,
    ],
)(_sc_body)


def kernel(x, pos_embedding):
    del x
    out = _sc_call(pos_embedding[:SEQ].reshape(1, ROW))
    return out.reshape(BATCH, SEQ, H_DIM)
